# baseline (speedup 1.0000x reference)
import functools

import jax
import jax.numpy as jnp
from jax import lax
from jax.experimental import pallas as pl
from jax.experimental.pallas import tpu as pltpu
from jax.experimental.pallas import tpu_sc as plsc

N = 10000
D = 128
E = 320000
NC, NS = 2, 16
NW = NC * NS
EPW = E // NW
CHUNK = 125
NCH = EPW // CHUNK
IB = 10
NG = NCH // IB
HGRP = EPW // 16

_sc_mesh = plsc.VectorSubcoreMesh(
    core_axis_name="c", subcore_axis_name="s", num_cores=NC, num_subcores=NS)


@functools.partial(
    pl.kernel,
    out_type=jax.ShapeDtypeStruct((NW, N), jnp.float32),
    mesh=_sc_mesh,
    compiler_params=pltpu.CompilerParams(needs_layout_passes=False),
    scratch_types=[
        pltpu.VMEM((EPW,), jnp.int32),
        pltpu.VMEM((N,), jnp.float32),
    ],
)
def _sc_degree(dst_hbm, zeros_hbm, out_hbm, dst_v, hist):
    c = lax.axis_index("c")
    s = lax.axis_index("s")
    wid = c * NS + s
    pltpu.sync_copy(dst_hbm.at[wid], dst_v)
    pltpu.sync_copy(zeros_hbm, hist)
    ones = jnp.ones((16,), jnp.float32)

    @pl.loop(0, HGRP)
    def _(g):
        idx = dst_v[pl.ds(g * 16, 16)]
        plsc.addupdate_scatter(hist, [idx], ones)

    pltpu.sync_copy(hist, out_hbm.at[wid])


@functools.partial(
    pl.kernel,
    out_type=jax.ShapeDtypeStruct((NC, N, D), jnp.float32),
    mesh=_sc_mesh,
    compiler_params=pltpu.CompilerParams(needs_layout_passes=False),
    scratch_types=[
        pltpu.VMEM((IB, 2, CHUNK), jnp.int32),
        pltpu.VMEM((IB, 2, CHUNK), jnp.int32),
        pltpu.VMEM((CHUNK, D), jnp.float32),
        pltpu.VMEM((CHUNK, D), jnp.float32),
        pltpu.VMEM_SHARED((N, D), jnp.float32),
        pltpu.SemaphoreType.DMA,
        pltpu.SemaphoreType.DMA,
        pltpu.SemaphoreType.DMA,
    ],
)
def _sc_edge_accum(idx_hbm, g_hbm, zeros_hbm, out_hbm,
                   idx0, idx1, buf0, buf1, acc, semi, sem0, sem1):
    c = lax.axis_index("c")
    s = lax.axis_index("s")
    wid = c * NS + s

    def idx_group(g):
        return idx_hbm.at[wid, pl.ds(g * IB, IB)]

    pltpu.async_copy(idx_group(0), idx0, semi)

    @pl.when(s < NS - 1)
    def _():
        pltpu.sync_copy(zeros_hbm.at[pl.ds(s * 624, 624)],
                        acc.at[pl.ds(s * 624, 624)])

    @pl.when(s == NS - 1)
    def _():
        pltpu.sync_copy(zeros_hbm.at[pl.ds(9360, 640)],
                        acc.at[pl.ds(9360, 640)])

    plsc.subcore_barrier()

    def run_group(idx_v):
        pltpu.async_copy(g_hbm.at[idx_v.at[0, 0]], buf0, sem0)

        @pl.loop(0, IB // 2)
        def _(k):
            j = k * 2
            pltpu.make_async_copy(g_hbm.at[idx_v.at[j, 0]], buf0, sem0).wait()
            pltpu.async_copy(g_hbm.at[idx_v.at[j + 1, 0]], buf1, sem1)
            pltpu.sync_copy(buf0, acc.at[idx_v.at[j, 1]], add=True)
            pltpu.make_async_copy(
                g_hbm.at[idx_v.at[j + 1, 0]], buf1, sem1).wait()

            @pl.when(j + 2 < IB)
            def _():
                pltpu.async_copy(g_hbm.at[idx_v.at[j + 2, 0]], buf0, sem0)

            pltpu.sync_copy(buf1, acc.at[idx_v.at[j + 1, 1]], add=True)

    @pl.loop(0, NG // 2)
    def _(i):
        g = i * 2
        pltpu.make_async_copy(idx_group(g), idx0, semi).wait()
        pltpu.async_copy(idx_group(g + 1), idx1, semi)
        run_group(idx0)
        pltpu.make_async_copy(idx_group(g + 1), idx1, semi).wait()

        @pl.when(g + 2 < NG)
        def _():
            pltpu.async_copy(idx_group(g + 2), idx0, semi)

        run_group(idx1)

    plsc.subcore_barrier()

    @pl.when(s < NS - 1)
    def _():
        pltpu.sync_copy(acc.at[pl.ds(s * 624, 624)],
                        out_hbm.at[c, pl.ds(s * 624, 624)])

    @pl.when(s == NS - 1)
    def _():
        pltpu.sync_copy(acc.at[pl.ds(9360, 640)],
                        out_hbm.at[c, pl.ds(9360, 640)])


_R = 1000
_GRID = N // _R


def _tc_h_body(x_ref, w_ref, h_ref):
    h_ref[...] = jnp.dot(
        x_ref[...], w_ref[...], preferred_element_type=jnp.float32)


def _tc_g_body(deg_ref, h_ref, g_ref, dinv_ref):
    dinv = lax.rsqrt(jnp.sum(deg_ref[...], axis=0) + 1.0)[:, None]
    g_ref[...] = h_ref[...] * dinv
    dinv_ref[...] = dinv


def _tc_mid_body(dinv_ref, acc_ref, g_ref, b_ref, w_ref, g2_ref):
    dinv = dinv_ref[...]
    z = (acc_ref[0] + acc_ref[1] + g_ref[...]) * dinv + b_ref[...]
    z = jnp.maximum(z, 0.0)
    g2_ref[...] = jnp.dot(
        z, w_ref[...], preferred_element_type=jnp.float32) * dinv


def _tc_out_body(dinv_ref, acc_ref, g_ref, b_ref, o_ref):
    dinv = dinv_ref[...]
    o_ref[...] = (acc_ref[0] + acc_ref[1] + g_ref[...]) * dinv + b_ref[...]


_dinv_spec = pl.BlockSpec((_R, 1), lambda i: (i, 0))
_row_spec = pl.BlockSpec((_R, D), lambda i: (i, 0))
_deg_spec = pl.BlockSpec((NW, _R), lambda i: (0, i))
_acc_spec = pl.BlockSpec((NC, _R, D), lambda i: (0, i, 0))
_w_spec = pl.BlockSpec((D, D), lambda i: (0, 0))
_b_spec = pl.BlockSpec((1, D), lambda i: (0, 0))
_out_nd = jax.ShapeDtypeStruct((N, D), jnp.float32)

_tc_h = pl.pallas_call(
    _tc_h_body, grid=(_GRID,),
    in_specs=[_row_spec, _w_spec],
    out_specs=_row_spec, out_shape=_out_nd)

_tc_g = pl.pallas_call(
    _tc_g_body,
    out_shape=[_out_nd, jax.ShapeDtypeStruct((N, 1), jnp.float32)])

_tc_mid = pl.pallas_call(
    _tc_mid_body, grid=(_GRID,),
    in_specs=[_dinv_spec, _acc_spec, _row_spec, _b_spec, _w_spec],
    out_specs=_row_spec, out_shape=_out_nd)

_tc_out = pl.pallas_call(
    _tc_out_body, grid=(_GRID,),
    in_specs=[_dinv_spec, _acc_spec, _row_spec, _b_spec],
    out_specs=_row_spec, out_shape=_out_nd)


def kernel(x, edge_index, W1, b1, W2, b2):
    src = edge_index[0].astype(jnp.int32).reshape(NW, NCH, CHUNK)
    dst = edge_index[1].astype(jnp.int32).reshape(NW, NCH, CHUNK)
    idx = jnp.stack([src, dst], axis=2)
    dst_flat = edge_index[1].astype(jnp.int32).reshape(NW, EPW)
    zeros_nd = jnp.zeros((N, D), jnp.float32)
    zeros_hist = jnp.zeros((N,), jnp.float32)

    deg_parts = _sc_degree(dst_flat, zeros_hist)
    h1 = _tc_h(x, W1)
    b1r = b1.reshape(1, D)
    b2r = b2.reshape(1, D)

    g1, dinv = _tc_g(deg_parts, h1)
    acc1 = _sc_edge_accum(idx, g1, zeros_nd)
    g2 = _tc_mid(dinv, acc1, g1, b1r, W2)
    acc2 = _sc_edge_accum(idx, g2, zeros_nd)
    out = _tc_out(dinv, acc2, g2, b2r)
    return out


# speedup vs baseline: 1.0005x; 1.0005x over previous
import functools

import jax
import jax.numpy as jnp
from jax import lax
from jax.experimental import pallas as pl
from jax.experimental.pallas import tpu as pltpu
from jax.experimental.pallas import tpu_sc as plsc

N = 10000
D = 128
E = 320000
NC, NS = 2, 16
NW = NC * NS
EPW = E // NW
CHUNK = 125
NCH = EPW // CHUNK
IB = 10
NG = NCH // IB
HGRP = EPW // 16

_sc_mesh = plsc.VectorSubcoreMesh(
    core_axis_name="c", subcore_axis_name="s", num_cores=NC, num_subcores=NS)


@functools.partial(
    pl.kernel,
    out_type=jax.ShapeDtypeStruct((NW, N), jnp.float32),
    mesh=_sc_mesh,
    compiler_params=pltpu.CompilerParams(needs_layout_passes=False),
    scratch_types=[
        pltpu.VMEM((EPW,), jnp.int32),
        pltpu.VMEM((N,), jnp.float32),
    ],
)
def _sc_degree(dst_hbm, zeros_hbm, out_hbm, dst_v, hist):
    c = lax.axis_index("c")
    s = lax.axis_index("s")
    wid = c * NS + s
    pltpu.sync_copy(dst_hbm.at[wid], dst_v)
    pltpu.sync_copy(zeros_hbm, hist)
    ones = jnp.ones((16,), jnp.float32)

    @pl.loop(0, HGRP)
    def _(g):
        idx = dst_v[pl.ds(g * 16, 16)]
        plsc.addupdate_scatter(hist, [idx], ones)

    pltpu.sync_copy(hist, out_hbm.at[wid])


@functools.partial(
    pl.kernel,
    out_type=jax.ShapeDtypeStruct((NC, N, D), jnp.float32),
    mesh=_sc_mesh,
    compiler_params=pltpu.CompilerParams(needs_layout_passes=False),
    scratch_types=[
        pltpu.VMEM((IB, 2, CHUNK), jnp.int32),
        pltpu.VMEM((IB, 2, CHUNK), jnp.int32),
        pltpu.VMEM((CHUNK, D), jnp.float32),
        pltpu.VMEM((CHUNK, D), jnp.float32),
        pltpu.VMEM_SHARED((N, D), jnp.float32),
        pltpu.SemaphoreType.DMA,
        pltpu.SemaphoreType.DMA,
        pltpu.SemaphoreType.DMA,
        pltpu.SemaphoreType.DMA,
        pltpu.SemaphoreType.DMA,
    ],
)
def _sc_edge_accum(idx_hbm, g_hbm, zeros_hbm, out_hbm,
                   idx0, idx1, buf0, buf1, acc, semi, sem0, sem1,
                   sems0, sems1):
    c = lax.axis_index("c")
    s = lax.axis_index("s")
    wid = c * NS + s

    def idx_group(g):
        return idx_hbm.at[wid, pl.ds(g * IB, IB)]

    pltpu.async_copy(idx_group(0), idx0, semi)

    @pl.when(s < NS - 1)
    def _():
        pltpu.sync_copy(zeros_hbm.at[pl.ds(s * 624, 624)],
                        acc.at[pl.ds(s * 624, 624)])

    @pl.when(s == NS - 1)
    def _():
        pltpu.sync_copy(zeros_hbm.at[pl.ds(9360, 640)],
                        acc.at[pl.ds(9360, 640)])

    plsc.subcore_barrier()

    def wait_scat1(idx_v):
        pltpu.make_async_copy(buf1, acc.at[idx_v.at[1, 1]], sems1).wait()

    def run_group(idx_v):
        pltpu.async_copy(g_hbm.at[idx_v.at[0, 0]], buf0, sem0)

        @pl.loop(0, IB // 2)
        def _(k):
            j = k * 2
            pltpu.make_async_copy(g_hbm.at[idx_v.at[j, 0]], buf0, sem0).wait()

            @pl.when(j >= 2)
            def _():
                wait_scat1(idx_v)

            pltpu.async_copy(g_hbm.at[idx_v.at[j + 1, 0]], buf1, sem1)
            pltpu.async_copy(buf0, acc.at[idx_v.at[j, 1]], sems0, add=True)
            pltpu.make_async_copy(
                g_hbm.at[idx_v.at[j + 1, 0]], buf1, sem1).wait()
            pltpu.make_async_copy(buf0, acc.at[idx_v.at[j, 1]], sems0).wait()

            @pl.when(j + 2 < IB)
            def _():
                pltpu.async_copy(g_hbm.at[idx_v.at[j + 2, 0]], buf0, sem0)

            pltpu.async_copy(buf1, acc.at[idx_v.at[j + 1, 1]], sems1, add=True)

    @pl.loop(0, NG // 2)
    def _(i):
        g = i * 2
        pltpu.make_async_copy(idx_group(g), idx0, semi).wait()
        pltpu.async_copy(idx_group(g + 1), idx1, semi)

        @pl.when(g > 0)
        def _():
            wait_scat1(idx0)

        run_group(idx0)
        pltpu.make_async_copy(idx_group(g + 1), idx1, semi).wait()

        @pl.when(g + 2 < NG)
        def _():
            pltpu.async_copy(idx_group(g + 2), idx0, semi)

        wait_scat1(idx1)
        run_group(idx1)

    wait_scat1(idx1)
    plsc.subcore_barrier()

    @pl.when(s < NS - 1)
    def _():
        pltpu.sync_copy(acc.at[pl.ds(s * 624, 624)],
                        out_hbm.at[c, pl.ds(s * 624, 624)])

    @pl.when(s == NS - 1)
    def _():
        pltpu.sync_copy(acc.at[pl.ds(9360, 640)],
                        out_hbm.at[c, pl.ds(9360, 640)])


_R = 1000
_GRID = N // _R


def _tc_h_body(x_ref, w_ref, h_ref):
    h_ref[...] = jnp.dot(
        x_ref[...], w_ref[...], preferred_element_type=jnp.float32)


def _tc_g_body(deg_ref, h_ref, g_ref, dinv_ref):
    dinv = lax.rsqrt(jnp.sum(deg_ref[...], axis=0) + 1.0)[:, None]
    g_ref[...] = h_ref[...] * dinv
    dinv_ref[...] = dinv


def _tc_mid_body(dinv_ref, acc_ref, g_ref, b_ref, w_ref, g2_ref):
    dinv = dinv_ref[...]
    z = (acc_ref[0] + acc_ref[1] + g_ref[...]) * dinv + b_ref[...]
    z = jnp.maximum(z, 0.0)
    g2_ref[...] = jnp.dot(
        z, w_ref[...], preferred_element_type=jnp.float32) * dinv


def _tc_out_body(dinv_ref, acc_ref, g_ref, b_ref, o_ref):
    dinv = dinv_ref[...]
    o_ref[...] = (acc_ref[0] + acc_ref[1] + g_ref[...]) * dinv + b_ref[...]


_dinv_spec = pl.BlockSpec((_R, 1), lambda i: (i, 0))
_row_spec = pl.BlockSpec((_R, D), lambda i: (i, 0))
_deg_spec = pl.BlockSpec((NW, _R), lambda i: (0, i))
_acc_spec = pl.BlockSpec((NC, _R, D), lambda i: (0, i, 0))
_w_spec = pl.BlockSpec((D, D), lambda i: (0, 0))
_b_spec = pl.BlockSpec((1, D), lambda i: (0, 0))
_out_nd = jax.ShapeDtypeStruct((N, D), jnp.float32)

_tc_h = pl.pallas_call(
    _tc_h_body, grid=(_GRID,),
    in_specs=[_row_spec, _w_spec],
    out_specs=_row_spec, out_shape=_out_nd)

_tc_g = pl.pallas_call(
    _tc_g_body,
    out_shape=[_out_nd, jax.ShapeDtypeStruct((N, 1), jnp.float32)])

_tc_mid = pl.pallas_call(
    _tc_mid_body, grid=(_GRID,),
    in_specs=[_dinv_spec, _acc_spec, _row_spec, _b_spec, _w_spec],
    out_specs=_row_spec, out_shape=_out_nd)

_tc_out = pl.pallas_call(
    _tc_out_body, grid=(_GRID,),
    in_specs=[_dinv_spec, _acc_spec, _row_spec, _b_spec],
    out_specs=_row_spec, out_shape=_out_nd)


def kernel(x, edge_index, W1, b1, W2, b2):
    src = edge_index[0].astype(jnp.int32).reshape(NW, NCH, CHUNK)
    dst = edge_index[1].astype(jnp.int32).reshape(NW, NCH, CHUNK)
    idx = jnp.stack([src, dst], axis=2)
    dst_flat = edge_index[1].astype(jnp.int32).reshape(NW, EPW)
    zeros_nd = jnp.zeros((N, D), jnp.float32)
    zeros_hist = jnp.zeros((N,), jnp.float32)

    deg_parts = _sc_degree(dst_flat, zeros_hist)
    h1 = _tc_h(x, W1)
    b1r = b1.reshape(1, D)
    b2r = b2.reshape(1, D)

    g1, dinv = _tc_g(deg_parts, h1)
    acc1 = _sc_edge_accum(idx, g1, zeros_nd)
    g2 = _tc_mid(dinv, acc1, g1, b1r, W2)
    acc2 = _sc_edge_accum(idx, g2, zeros_nd)
    out = _tc_out(dinv, acc2, g2, b2r)
    return out


# speedup vs baseline: 1.0023x; 1.0018x over previous
import functools

import jax
import jax.numpy as jnp
from jax import lax
from jax.experimental import pallas as pl
from jax.experimental.pallas import tpu as pltpu
from jax.experimental.pallas import tpu_sc as plsc

N = 10000
D = 128
E = 320000
NC, NS = 2, 16
NW = NC * NS
EPW = E // NW
CHUNK = 125
NCH = EPW // CHUNK
IB = 10
NG = NCH // IB
HGRP = EPW // 16

_sc_mesh = plsc.VectorSubcoreMesh(
    core_axis_name="c", subcore_axis_name="s", num_cores=NC, num_subcores=NS)


@functools.partial(
    pl.kernel,
    out_type=jax.ShapeDtypeStruct((NW, N), jnp.float32),
    mesh=_sc_mesh,
    compiler_params=pltpu.CompilerParams(needs_layout_passes=False),
    scratch_types=[
        pltpu.VMEM((EPW,), jnp.int32),
        pltpu.VMEM((N,), jnp.float32),
    ],
)
def _sc_degree(dst_hbm, zeros_hbm, out_hbm, dst_v, hist):
    c = lax.axis_index("c")
    s = lax.axis_index("s")
    wid = c * NS + s
    pltpu.sync_copy(dst_hbm.at[wid], dst_v)
    pltpu.sync_copy(zeros_hbm, hist)
    ones = jnp.ones((16,), jnp.float32)

    @pl.loop(0, HGRP)
    def _(g):
        idx = dst_v[pl.ds(g * 16, 16)]
        plsc.addupdate_scatter(hist, [idx], ones)

    pltpu.sync_copy(hist, out_hbm.at[wid])


@functools.partial(
    pl.kernel,
    out_type=jax.ShapeDtypeStruct((NC, N, D), jnp.float32),
    mesh=_sc_mesh,
    compiler_params=pltpu.CompilerParams(needs_layout_passes=False),
    scratch_types=[
        pltpu.VMEM((IB, 2, CHUNK), jnp.int32),
        pltpu.VMEM((IB, 2, CHUNK), jnp.int32),
        pltpu.VMEM((CHUNK, D), jnp.float32),
        pltpu.VMEM((CHUNK, D), jnp.float32),
        pltpu.VMEM_SHARED((N, D), jnp.float32),
        pltpu.SemaphoreType.DMA,
        pltpu.SemaphoreType.DMA,
        pltpu.SemaphoreType.DMA,
        pltpu.SemaphoreType.DMA,
        pltpu.SemaphoreType.DMA,
    ],
)
def _sc_edge_accum(idx_hbm, g_hbm, zeros_hbm, out_hbm,
                   idx0, idx1, buf0, buf1, acc, semi, sem0, sem1,
                   sems0, sems1):
    c = lax.axis_index("c")
    s = lax.axis_index("s")
    wid = c * NS + s

    def idx_group(g):
        return idx_hbm.at[wid, pl.ds(g * IB, IB)]

    pltpu.async_copy(idx_group(0), idx0, semi)

    @pl.when(s < NS - 1)
    def _():
        pltpu.sync_copy(zeros_hbm.at[pl.ds(s * 624, 624)],
                        acc.at[pl.ds(s * 624, 624)])

    @pl.when(s == NS - 1)
    def _():
        pltpu.sync_copy(zeros_hbm.at[pl.ds(9360, 640)],
                        acc.at[pl.ds(9360, 640)])

    plsc.subcore_barrier()

    def wait_scat1(idx_v):
        pltpu.make_async_copy(buf1, acc.at[idx_v.at[1, 1]], sems1).wait()

    def run_group(idx_v):
        pltpu.async_copy(g_hbm.at[idx_v.at[0, 0]], buf0, sem0)

        @pl.loop(0, IB // 2)
        def _(k):
            j = k * 2
            pltpu.make_async_copy(g_hbm.at[idx_v.at[j, 0]], buf0, sem0).wait()

            @pl.when(j >= 2)
            def _():
                wait_scat1(idx_v)

            pltpu.async_copy(g_hbm.at[idx_v.at[j + 1, 0]], buf1, sem1)
            pltpu.async_copy(buf0, acc.at[idx_v.at[j, 1]], sems0, add=True)
            pltpu.make_async_copy(
                g_hbm.at[idx_v.at[j + 1, 0]], buf1, sem1).wait()
            pltpu.make_async_copy(buf0, acc.at[idx_v.at[j, 1]], sems0).wait()

            @pl.when(j + 2 < IB)
            def _():
                pltpu.async_copy(g_hbm.at[idx_v.at[j + 2, 0]], buf0, sem0)

            pltpu.async_copy(buf1, acc.at[idx_v.at[j + 1, 1]], sems1, add=True)

    @pl.loop(0, NG // 2)
    def _(i):
        g = i * 2
        pltpu.make_async_copy(idx_group(g), idx0, semi).wait()
        pltpu.async_copy(idx_group(g + 1), idx1, semi)

        @pl.when(g > 0)
        def _():
            wait_scat1(idx0)

        run_group(idx0)
        pltpu.make_async_copy(idx_group(g + 1), idx1, semi).wait()

        @pl.when(g + 2 < NG)
        def _():
            pltpu.async_copy(idx_group(g + 2), idx0, semi)

        wait_scat1(idx1)
        run_group(idx1)

    wait_scat1(idx1)
    plsc.subcore_barrier()

    @pl.when(s < NS - 1)
    def _():
        pltpu.sync_copy(acc.at[pl.ds(s * 624, 624)],
                        out_hbm.at[c, pl.ds(s * 624, 624)])

    @pl.when(s == NS - 1)
    def _():
        pltpu.sync_copy(acc.at[pl.ds(9360, 640)],
                        out_hbm.at[c, pl.ds(9360, 640)])


_R = 1000
_GRID = N // _R


def _tc_hg_body(deg_ref, x_ref, w_ref, g_ref, dinv_ref):
    dinv = lax.rsqrt(jnp.sum(deg_ref[...], axis=1) + 1.0)[:, None]
    g_ref[...] = jnp.dot(
        x_ref[...], w_ref[...], preferred_element_type=jnp.float32) * dinv
    dinv_ref[...] = dinv


def _tc_mid_body(dinv_ref, acc_ref, g_ref, b_ref, w_ref, g2_ref):
    dinv = dinv_ref[...]
    z = (acc_ref[0] + acc_ref[1] + g_ref[...]) * dinv + b_ref[...]
    z = jnp.maximum(z, 0.0)
    g2_ref[...] = jnp.dot(
        z, w_ref[...], preferred_element_type=jnp.float32) * dinv


def _tc_out_body(dinv_ref, acc_ref, g_ref, b_ref, o_ref):
    dinv = dinv_ref[...]
    o_ref[...] = (acc_ref[0] + acc_ref[1] + g_ref[...]) * dinv + b_ref[...]


_dinv_spec = pl.BlockSpec((_R, 1), lambda i: (i, 0))
_row_spec = pl.BlockSpec((_R, D), lambda i: (i, 0))
_deg_spec = pl.BlockSpec((_R, NW), lambda i: (i, 0))
_acc_spec = pl.BlockSpec((NC, _R, D), lambda i: (0, i, 0))
_w_spec = pl.BlockSpec((D, D), lambda i: (0, 0))
_b_spec = pl.BlockSpec((1, D), lambda i: (0, 0))
_out_nd = jax.ShapeDtypeStruct((N, D), jnp.float32)

_tc_hg = pl.pallas_call(
    _tc_hg_body, grid=(_GRID,),
    in_specs=[_deg_spec, _row_spec, _w_spec],
    out_specs=[_row_spec, _dinv_spec],
    out_shape=[_out_nd, jax.ShapeDtypeStruct((N, 1), jnp.float32)])

_tc_mid = pl.pallas_call(
    _tc_mid_body, grid=(_GRID,),
    in_specs=[_dinv_spec, _acc_spec, _row_spec, _b_spec, _w_spec],
    out_specs=_row_spec, out_shape=_out_nd)

_tc_out = pl.pallas_call(
    _tc_out_body, grid=(_GRID,),
    in_specs=[_dinv_spec, _acc_spec, _row_spec, _b_spec],
    out_specs=_row_spec, out_shape=_out_nd)


def kernel(x, edge_index, W1, b1, W2, b2):
    src = edge_index[0].astype(jnp.int32).reshape(NW, NCH, CHUNK)
    dst = edge_index[1].astype(jnp.int32).reshape(NW, NCH, CHUNK)
    idx = jnp.stack([src, dst], axis=2)
    dst_flat = edge_index[1].astype(jnp.int32).reshape(NW, EPW)
    zeros_nd = jnp.zeros((N, D), jnp.float32)
    zeros_hist = jnp.zeros((N,), jnp.float32)

    deg_parts = _sc_degree(dst_flat, zeros_hist)
    b1r = b1.reshape(1, D)
    b2r = b2.reshape(1, D)

    g1, dinv = _tc_hg(deg_parts.T, x, W1)
    acc1 = _sc_edge_accum(idx, g1, zeros_nd)
    g2 = _tc_mid(dinv, acc1, g1, b1r, W2)
    acc2 = _sc_edge_accum(idx, g2, zeros_nd)
    out = _tc_out(dinv, acc2, g2, b2r)
    return out


# speedup vs baseline: 1.0290x; 1.0266x over previous
import functools

import jax
import jax.numpy as jnp
from jax import lax
from jax.experimental import pallas as pl
from jax.experimental.pallas import tpu as pltpu
from jax.experimental.pallas import tpu_sc as plsc

N = 10000
D = 128
E = 320000
NC, NS = 2, 16
NW = NC * NS
EPW = E // NW
CHUNK = 125
NCH = EPW // CHUNK
IB = 10
NG = NCH // IB
HGRP = EPW // 16

_sc_mesh = plsc.VectorSubcoreMesh(
    core_axis_name="c", subcore_axis_name="s", num_cores=NC, num_subcores=NS)


@functools.partial(
    pl.kernel,
    out_type=jax.ShapeDtypeStruct((NW, N), jnp.float32),
    mesh=_sc_mesh,
    compiler_params=pltpu.CompilerParams(needs_layout_passes=False),
    scratch_types=[
        pltpu.VMEM((EPW,), jnp.int32),
        pltpu.VMEM((N,), jnp.float32),
    ],
)
def _sc_degree(dst_hbm, out_hbm, dst_v, hist):
    c = lax.axis_index("c")
    s = lax.axis_index("s")
    wid = c * NS + s
    pltpu.sync_copy(dst_hbm.at[wid], dst_v)
    zeros16 = jnp.zeros((16,), jnp.float32)

    @pl.loop(0, N // 16)
    def _(g):
        hist[pl.ds(g * 16, 16)] = zeros16

    ones = jnp.ones((16,), jnp.float32)

    @pl.loop(0, HGRP)
    def _(g):
        idx = dst_v[pl.ds(g * 16, 16)]
        plsc.addupdate_scatter(hist, [idx], ones)

    pltpu.sync_copy(hist, out_hbm.at[wid])


@functools.partial(
    pl.kernel,
    out_type=jax.ShapeDtypeStruct((NC, N, D), jnp.float32),
    mesh=_sc_mesh,
    compiler_params=pltpu.CompilerParams(needs_layout_passes=False),
    scratch_types=[
        pltpu.VMEM((IB, 2, CHUNK), jnp.int32),
        pltpu.VMEM((IB, 2, CHUNK), jnp.int32),
        pltpu.VMEM((CHUNK, D), jnp.float32),
        pltpu.VMEM((CHUNK, D), jnp.float32),
        pltpu.VMEM_SHARED((N, D), jnp.float32),
        pltpu.SemaphoreType.DMA,
        pltpu.SemaphoreType.DMA,
        pltpu.SemaphoreType.DMA,
        pltpu.SemaphoreType.DMA,
        pltpu.SemaphoreType.DMA,
    ],
)
def _sc_edge_accum(idx_hbm, g_hbm, out_hbm,
                   idx0, idx1, buf0, buf1, acc, semi, sem0, sem1,
                   sems0, sems1):
    c = lax.axis_index("c")
    s = lax.axis_index("s")
    wid = c * NS + s

    def idx_group(g):
        return idx_hbm.at[wid, pl.ds(g * IB, IB)]

    pltpu.async_copy(idx_group(0), idx0, semi)

    zeros16 = jnp.zeros((16,), jnp.float32)

    @pl.loop(0, 40)
    def _(r):
        @pl.loop(0, D // 16)
        def _(k):
            buf0[r, pl.ds(k * 16, 16)] = zeros16

    @pl.loop(0, 15)
    def _(k):
        pltpu.sync_copy(buf0.at[pl.ds(0, 40)],
                        acc.at[pl.ds(s * 624 + k * 40, 40)])

    pltpu.sync_copy(buf0.at[pl.ds(0, 24)], acc.at[pl.ds(s * 624 + 600, 24)])

    @pl.when(s == NS - 1)
    def _():
        pltpu.sync_copy(buf0.at[pl.ds(0, 16)], acc.at[pl.ds(9984, 16)])

    plsc.subcore_barrier()

    def wait_scat1(idx_v):
        pltpu.make_async_copy(buf1, acc.at[idx_v.at[1, 1]], sems1).wait()

    def run_group(idx_v):
        pltpu.async_copy(g_hbm.at[idx_v.at[0, 0]], buf0, sem0)

        @pl.loop(0, IB // 2)
        def _(k):
            j = k * 2
            pltpu.make_async_copy(g_hbm.at[idx_v.at[j, 0]], buf0, sem0).wait()

            @pl.when(j >= 2)
            def _():
                wait_scat1(idx_v)

            pltpu.async_copy(g_hbm.at[idx_v.at[j + 1, 0]], buf1, sem1)
            pltpu.async_copy(buf0, acc.at[idx_v.at[j, 1]], sems0, add=True)
            pltpu.make_async_copy(
                g_hbm.at[idx_v.at[j + 1, 0]], buf1, sem1).wait()
            pltpu.make_async_copy(buf0, acc.at[idx_v.at[j, 1]], sems0).wait()

            @pl.when(j + 2 < IB)
            def _():
                pltpu.async_copy(g_hbm.at[idx_v.at[j + 2, 0]], buf0, sem0)

            pltpu.async_copy(buf1, acc.at[idx_v.at[j + 1, 1]], sems1, add=True)

    @pl.loop(0, NG // 2)
    def _(i):
        g = i * 2
        pltpu.make_async_copy(idx_group(g), idx0, semi).wait()
        pltpu.async_copy(idx_group(g + 1), idx1, semi)

        @pl.when(g > 0)
        def _():
            wait_scat1(idx0)

        run_group(idx0)
        pltpu.make_async_copy(idx_group(g + 1), idx1, semi).wait()

        @pl.when(g + 2 < NG)
        def _():
            pltpu.async_copy(idx_group(g + 2), idx0, semi)

        wait_scat1(idx1)
        run_group(idx1)

    wait_scat1(idx1)
    plsc.subcore_barrier()

    @pl.when(s < NS - 1)
    def _():
        pltpu.sync_copy(acc.at[pl.ds(s * 624, 624)],
                        out_hbm.at[c, pl.ds(s * 624, 624)])

    @pl.when(s == NS - 1)
    def _():
        pltpu.sync_copy(acc.at[pl.ds(9360, 640)],
                        out_hbm.at[c, pl.ds(9360, 640)])


_R = 1000
_GRID = N // _R


def _tc_hg_body(deg_ref, x_ref, w_ref, g_ref, dinv_ref):
    dinv = lax.rsqrt(jnp.sum(deg_ref[...], axis=1) + 1.0)[:, None]
    g_ref[...] = jnp.dot(
        x_ref[...], w_ref[...], preferred_element_type=jnp.float32) * dinv
    dinv_ref[...] = dinv


def _tc_mid_body(dinv_ref, acc_ref, g_ref, b_ref, w_ref, g2_ref):
    dinv = dinv_ref[...]
    z = (acc_ref[0] + acc_ref[1] + g_ref[...]) * dinv + b_ref[...]
    z = jnp.maximum(z, 0.0)
    g2_ref[...] = jnp.dot(
        z, w_ref[...], preferred_element_type=jnp.float32) * dinv


def _tc_out_body(dinv_ref, acc_ref, g_ref, b_ref, o_ref):
    dinv = dinv_ref[...]
    o_ref[...] = (acc_ref[0] + acc_ref[1] + g_ref[...]) * dinv + b_ref[...]


_dinv_spec = pl.BlockSpec((_R, 1), lambda i: (i, 0))
_row_spec = pl.BlockSpec((_R, D), lambda i: (i, 0))
_deg_spec = pl.BlockSpec((_R, NW), lambda i: (i, 0))
_acc_spec = pl.BlockSpec((NC, _R, D), lambda i: (0, i, 0))
_w_spec = pl.BlockSpec((D, D), lambda i: (0, 0))
_b_spec = pl.BlockSpec((1, D), lambda i: (0, 0))
_out_nd = jax.ShapeDtypeStruct((N, D), jnp.float32)

_tc_hg = pl.pallas_call(
    _tc_hg_body, grid=(_GRID,),
    in_specs=[_deg_spec, _row_spec, _w_spec],
    out_specs=[_row_spec, _dinv_spec],
    out_shape=[_out_nd, jax.ShapeDtypeStruct((N, 1), jnp.float32)])

_tc_mid = pl.pallas_call(
    _tc_mid_body, grid=(_GRID,),
    in_specs=[_dinv_spec, _acc_spec, _row_spec, _b_spec, _w_spec],
    out_specs=_row_spec, out_shape=_out_nd)

_tc_out = pl.pallas_call(
    _tc_out_body, grid=(_GRID,),
    in_specs=[_dinv_spec, _acc_spec, _row_spec, _b_spec],
    out_specs=_row_spec, out_shape=_out_nd)


def kernel(x, edge_index, W1, b1, W2, b2):
    src = edge_index[0].astype(jnp.int32).reshape(NW, NCH, CHUNK)
    dst = edge_index[1].astype(jnp.int32).reshape(NW, NCH, CHUNK)
    idx = jnp.stack([src, dst], axis=2)
    dst_flat = edge_index[1].astype(jnp.int32).reshape(NW, EPW)

    deg_parts = _sc_degree(dst_flat)
    b1r = b1.reshape(1, D)
    b2r = b2.reshape(1, D)

    g1, dinv = _tc_hg(deg_parts.T, x, W1)
    acc1 = _sc_edge_accum(idx, g1)
    g2 = _tc_mid(dinv, acc1, g1, b1r, W2)
    acc2 = _sc_edge_accum(idx, g2)
    out = _tc_out(dinv, acc2, g2, b2r)
    return out


# speedup vs baseline: 1.0341x; 1.0050x over previous
import functools

import jax
import jax.numpy as jnp
from jax import lax
from jax.experimental import pallas as pl
from jax.experimental.pallas import tpu as pltpu
from jax.experimental.pallas import tpu_sc as plsc

N = 10000
D = 128
E = 320000
NC, NS = 2, 16
NW = NC * NS
EPW = E // NW
CHUNK = 125
NCH = EPW // CHUNK
IB = 10
NG = NCH // IB
HGRP = EPW // 16

_sc_mesh = plsc.VectorSubcoreMesh(
    core_axis_name="c", subcore_axis_name="s", num_cores=NC, num_subcores=NS)


@functools.partial(
    pl.kernel,
    out_type=jax.ShapeDtypeStruct((NW, N), jnp.float32),
    mesh=_sc_mesh,
    compiler_params=pltpu.CompilerParams(needs_layout_passes=False),
    scratch_types=[
        pltpu.VMEM((EPW,), jnp.int32),
        pltpu.VMEM((N,), jnp.float32),
        pltpu.SemaphoreType.DMA,
    ],
)
def _sc_degree(dst_hbm, out_hbm, dst_v, hist, sem):
    c = lax.axis_index("c")
    s = lax.axis_index("s")
    wid = c * NS + s
    pltpu.async_copy(dst_hbm.at[wid], dst_v, sem)
    zeros16 = jnp.zeros((16,), jnp.float32)

    @pl.loop(0, N // 16)
    def _(g):
        hist[pl.ds(g * 16, 16)] = zeros16

    pltpu.make_async_copy(dst_hbm.at[wid], dst_v, sem).wait()
    ones = jnp.ones((16,), jnp.float32)

    @pl.loop(0, HGRP)
    def _(g):
        idx = dst_v[pl.ds(g * 16, 16)]
        plsc.addupdate_scatter(hist, [idx], ones)

    pltpu.sync_copy(hist, out_hbm.at[wid])


@functools.partial(
    pl.kernel,
    out_type=jax.ShapeDtypeStruct((NC, N, D), jnp.float32),
    mesh=_sc_mesh,
    compiler_params=pltpu.CompilerParams(needs_layout_passes=False),
    scratch_types=[
        pltpu.VMEM((IB, 2, CHUNK), jnp.int32),
        pltpu.VMEM((IB, 2, CHUNK), jnp.int32),
        pltpu.VMEM((CHUNK, D), jnp.float32),
        pltpu.VMEM((CHUNK, D), jnp.float32),
        pltpu.VMEM_SHARED((N, D), jnp.float32),
        pltpu.SemaphoreType.DMA,
        pltpu.SemaphoreType.DMA,
        pltpu.SemaphoreType.DMA,
        pltpu.SemaphoreType.DMA,
        pltpu.SemaphoreType.DMA,
    ],
)
def _sc_edge_accum(idx_hbm, g_hbm, out_hbm,
                   idx0, idx1, buf0, buf1, acc, semi, sem0, sem1,
                   sems0, sems1):
    c = lax.axis_index("c")
    s = lax.axis_index("s")
    wid = c * NS + s

    def idx_group(g):
        return idx_hbm.at[wid, pl.ds(g * IB, IB)]

    pltpu.async_copy(idx_group(0), idx0, semi)

    zeros16 = jnp.zeros((16,), jnp.float32)

    @pl.loop(0, 40)
    def _(r):
        @pl.loop(0, D // 16)
        def _(k):
            buf0[r, pl.ds(k * 16, 16)] = zeros16

    @pl.loop(0, 15)
    def _(k):
        pltpu.sync_copy(buf0.at[pl.ds(0, 40)],
                        acc.at[pl.ds(s * 624 + k * 40, 40)])

    pltpu.sync_copy(buf0.at[pl.ds(0, 24)], acc.at[pl.ds(s * 624 + 600, 24)])

    @pl.when(s == NS - 1)
    def _():
        pltpu.sync_copy(buf0.at[pl.ds(0, 16)], acc.at[pl.ds(9984, 16)])

    plsc.subcore_barrier()

    def wait_scat1(idx_v):
        pltpu.make_async_copy(buf1, acc.at[idx_v.at[1, 1]], sems1).wait()

    def run_group(idx_v):
        pltpu.async_copy(g_hbm.at[idx_v.at[0, 0]], buf0, sem0)

        @pl.loop(0, IB // 2)
        def _(k):
            j = k * 2
            pltpu.make_async_copy(g_hbm.at[idx_v.at[j, 0]], buf0, sem0).wait()

            @pl.when(j >= 2)
            def _():
                wait_scat1(idx_v)

            pltpu.async_copy(g_hbm.at[idx_v.at[j + 1, 0]], buf1, sem1)
            pltpu.async_copy(buf0, acc.at[idx_v.at[j, 1]], sems0, add=True)
            pltpu.make_async_copy(
                g_hbm.at[idx_v.at[j + 1, 0]], buf1, sem1).wait()
            pltpu.make_async_copy(buf0, acc.at[idx_v.at[j, 1]], sems0).wait()

            @pl.when(j + 2 < IB)
            def _():
                pltpu.async_copy(g_hbm.at[idx_v.at[j + 2, 0]], buf0, sem0)

            pltpu.async_copy(buf1, acc.at[idx_v.at[j + 1, 1]], sems1, add=True)

    @pl.loop(0, NG // 2)
    def _(i):
        g = i * 2
        pltpu.make_async_copy(idx_group(g), idx0, semi).wait()
        pltpu.async_copy(idx_group(g + 1), idx1, semi)

        @pl.when(g > 0)
        def _():
            wait_scat1(idx0)

        run_group(idx0)
        pltpu.make_async_copy(idx_group(g + 1), idx1, semi).wait()

        @pl.when(g + 2 < NG)
        def _():
            pltpu.async_copy(idx_group(g + 2), idx0, semi)

        wait_scat1(idx1)
        run_group(idx1)

    wait_scat1(idx1)
    plsc.subcore_barrier()

    @pl.when(s < NS - 1)
    def _():
        pltpu.sync_copy(acc.at[pl.ds(s * 624, 624)],
                        out_hbm.at[c, pl.ds(s * 624, 624)])

    @pl.when(s == NS - 1)
    def _():
        pltpu.sync_copy(acc.at[pl.ds(9360, 640)],
                        out_hbm.at[c, pl.ds(9360, 640)])


_R = 1000
_GRID = N // _R


def _tc_hg_body(deg_ref, x_ref, w_ref, g_ref, dinv_ref):
    dinv = lax.rsqrt(jnp.sum(deg_ref[...], axis=1) + 1.0)[:, None]
    g_ref[...] = jnp.dot(
        x_ref[...], w_ref[...], preferred_element_type=jnp.float32) * dinv
    dinv_ref[...] = dinv


def _tc_mid_body(dinv_ref, acc_ref, g_ref, b_ref, w_ref, g2_ref):
    dinv = dinv_ref[...]
    z = (acc_ref[0] + acc_ref[1] + g_ref[...]) * dinv + b_ref[...]
    z = jnp.maximum(z, 0.0)
    g2_ref[...] = jnp.dot(
        z, w_ref[...], preferred_element_type=jnp.float32) * dinv


def _tc_out_body(dinv_ref, acc_ref, g_ref, b_ref, o_ref):
    dinv = dinv_ref[...]
    o_ref[...] = (acc_ref[0] + acc_ref[1] + g_ref[...]) * dinv + b_ref[...]


_dinv_spec = pl.BlockSpec((_R, 1), lambda i: (i, 0))
_row_spec = pl.BlockSpec((_R, D), lambda i: (i, 0))
_deg_spec = pl.BlockSpec((_R, NW), lambda i: (i, 0))
_acc_spec = pl.BlockSpec((NC, _R, D), lambda i: (0, i, 0))
_w_spec = pl.BlockSpec((D, D), lambda i: (0, 0))
_b_spec = pl.BlockSpec((1, D), lambda i: (0, 0))
_out_nd = jax.ShapeDtypeStruct((N, D), jnp.float32)

_tc_hg = pl.pallas_call(
    _tc_hg_body, grid=(_GRID,),
    in_specs=[_deg_spec, _row_spec, _w_spec],
    out_specs=[_row_spec, _dinv_spec],
    out_shape=[_out_nd, jax.ShapeDtypeStruct((N, 1), jnp.float32)])

_tc_mid = pl.pallas_call(
    _tc_mid_body, grid=(_GRID,),
    in_specs=[_dinv_spec, _acc_spec, _row_spec, _b_spec, _w_spec],
    out_specs=_row_spec, out_shape=_out_nd)

_tc_out = pl.pallas_call(
    _tc_out_body, grid=(_GRID,),
    in_specs=[_dinv_spec, _acc_spec, _row_spec, _b_spec],
    out_specs=_row_spec, out_shape=_out_nd)


def kernel(x, edge_index, W1, b1, W2, b2):
    src = edge_index[0].astype(jnp.int32).reshape(NW, NCH, CHUNK)
    dst = edge_index[1].astype(jnp.int32).reshape(NW, NCH, CHUNK)
    idx = jnp.stack([src, dst], axis=2)
    dst_flat = edge_index[1].astype(jnp.int32).reshape(NW, EPW)

    deg_parts = _sc_degree(dst_flat)
    b1r = b1.reshape(1, D)
    b2r = b2.reshape(1, D)

    g1, dinv = _tc_hg(deg_parts.T, x, W1)
    acc1 = _sc_edge_accum(idx, g1)
    g2 = _tc_mid(dinv, acc1, g1, b1r, W2)
    acc2 = _sc_edge_accum(idx, g2)
    out = _tc_out(dinv, acc2, g2, b2r)
    return out


# speedup vs baseline: 1.0408x; 1.0065x over previous
import functools

import jax
import jax.numpy as jnp
from jax import lax
from jax.experimental import pallas as pl
from jax.experimental.pallas import tpu as pltpu
from jax.experimental.pallas import tpu_sc as plsc

N = 10000
D = 128
E = 320000
NC, NS = 2, 16
NW = NC * NS
EPW = E // NW
CHUNK = 125
NCH = EPW // CHUNK
IB = 20
NG = NCH // IB
HGRP = EPW // 16

_sc_mesh = plsc.VectorSubcoreMesh(
    core_axis_name="c", subcore_axis_name="s", num_cores=NC, num_subcores=NS)


@functools.partial(
    pl.kernel,
    out_type=jax.ShapeDtypeStruct((NW, N), jnp.float32),
    mesh=_sc_mesh,
    compiler_params=pltpu.CompilerParams(needs_layout_passes=False),
    scratch_types=[
        pltpu.VMEM((EPW,), jnp.int32),
        pltpu.VMEM((N,), jnp.float32),
        pltpu.SemaphoreType.DMA,
    ],
)
def _sc_degree(dst_hbm, out_hbm, dst_v, hist, sem):
    c = lax.axis_index("c")
    s = lax.axis_index("s")
    wid = c * NS + s
    pltpu.async_copy(dst_hbm.at[wid], dst_v, sem)
    zeros16 = jnp.zeros((16,), jnp.float32)

    @pl.loop(0, N // 16)
    def _(g):
        hist[pl.ds(g * 16, 16)] = zeros16

    pltpu.make_async_copy(dst_hbm.at[wid], dst_v, sem).wait()
    ones = jnp.ones((16,), jnp.float32)

    @pl.loop(0, HGRP)
    def _(g):
        idx = dst_v[pl.ds(g * 16, 16)]
        plsc.addupdate_scatter(hist, [idx], ones)

    pltpu.sync_copy(hist, out_hbm.at[wid])


@functools.partial(
    pl.kernel,
    out_type=jax.ShapeDtypeStruct((NC, N, D), jnp.float32),
    mesh=_sc_mesh,
    compiler_params=pltpu.CompilerParams(needs_layout_passes=False),
    scratch_types=[
        pltpu.VMEM((IB, 2, CHUNK), jnp.int32),
        pltpu.VMEM((IB, 2, CHUNK), jnp.int32),
        pltpu.VMEM((CHUNK, D), jnp.float32),
        pltpu.VMEM((CHUNK, D), jnp.float32),
        pltpu.VMEM_SHARED((N, D), jnp.float32),
        pltpu.SemaphoreType.DMA,
        pltpu.SemaphoreType.DMA,
        pltpu.SemaphoreType.DMA,
        pltpu.SemaphoreType.DMA,
        pltpu.SemaphoreType.DMA,
    ],
)
def _sc_edge_accum(idx_hbm, g_hbm, out_hbm,
                   idx0, idx1, buf0, buf1, acc, semi, sem0, sem1,
                   sems0, sems1):
    c = lax.axis_index("c")
    s = lax.axis_index("s")
    wid = c * NS + s

    def idx_group(g):
        return idx_hbm.at[wid, pl.ds(g * IB, IB)]

    pltpu.async_copy(idx_group(0), idx0, semi)

    zeros16 = jnp.zeros((16,), jnp.float32)

    @pl.loop(0, 40)
    def _(r):
        @pl.loop(0, D // 16)
        def _(k):
            buf0[r, pl.ds(k * 16, 16)] = zeros16

    @pl.loop(0, 15)
    def _(k):
        pltpu.sync_copy(buf0.at[pl.ds(0, 40)],
                        acc.at[pl.ds(s * 624 + k * 40, 40)])

    pltpu.sync_copy(buf0.at[pl.ds(0, 24)], acc.at[pl.ds(s * 624 + 600, 24)])

    @pl.when(s == NS - 1)
    def _():
        pltpu.sync_copy(buf0.at[pl.ds(0, 16)], acc.at[pl.ds(9984, 16)])

    plsc.subcore_barrier()

    def wait_scat1(idx_v):
        pltpu.make_async_copy(buf1, acc.at[idx_v.at[1, 1]], sems1).wait()

    def run_group(idx_v):
        pltpu.async_copy(g_hbm.at[idx_v.at[0, 0]], buf0, sem0)

        @pl.loop(0, IB // 2)
        def _(k):
            j = k * 2
            pltpu.make_async_copy(g_hbm.at[idx_v.at[j, 0]], buf0, sem0).wait()

            @pl.when(j >= 2)
            def _():
                wait_scat1(idx_v)

            pltpu.async_copy(g_hbm.at[idx_v.at[j + 1, 0]], buf1, sem1)
            pltpu.async_copy(buf0, acc.at[idx_v.at[j, 1]], sems0, add=True)
            pltpu.make_async_copy(
                g_hbm.at[idx_v.at[j + 1, 0]], buf1, sem1).wait()
            pltpu.make_async_copy(buf0, acc.at[idx_v.at[j, 1]], sems0).wait()

            @pl.when(j + 2 < IB)
            def _():
                pltpu.async_copy(g_hbm.at[idx_v.at[j + 2, 0]], buf0, sem0)

            pltpu.async_copy(buf1, acc.at[idx_v.at[j + 1, 1]], sems1, add=True)

    @pl.loop(0, NG // 2)
    def _(i):
        g = i * 2
        pltpu.make_async_copy(idx_group(g), idx0, semi).wait()
        pltpu.async_copy(idx_group(g + 1), idx1, semi)

        @pl.when(g > 0)
        def _():
            wait_scat1(idx0)

        run_group(idx0)
        pltpu.make_async_copy(idx_group(g + 1), idx1, semi).wait()

        @pl.when(g + 2 < NG)
        def _():
            pltpu.async_copy(idx_group(g + 2), idx0, semi)

        wait_scat1(idx1)
        run_group(idx1)

    wait_scat1(idx1)
    plsc.subcore_barrier()

    @pl.when(s < NS - 1)
    def _():
        pltpu.sync_copy(acc.at[pl.ds(s * 624, 624)],
                        out_hbm.at[c, pl.ds(s * 624, 624)])

    @pl.when(s == NS - 1)
    def _():
        pltpu.sync_copy(acc.at[pl.ds(9360, 640)],
                        out_hbm.at[c, pl.ds(9360, 640)])


_R = 1000
_GRID = N // _R


def _tc_hg_body(deg_ref, x_ref, w_ref, g_ref, dinv_ref):
    dinv = lax.rsqrt(jnp.sum(deg_ref[...], axis=1) + 1.0)[:, None]
    g_ref[...] = jnp.dot(
        x_ref[...], w_ref[...], preferred_element_type=jnp.float32) * dinv
    dinv_ref[...] = dinv


def _tc_mid_body(dinv_ref, acc_ref, g_ref, b_ref, w_ref, g2_ref):
    dinv = dinv_ref[...]
    z = (acc_ref[0] + acc_ref[1] + g_ref[...]) * dinv + b_ref[...]
    z = jnp.maximum(z, 0.0)
    g2_ref[...] = jnp.dot(
        z, w_ref[...], preferred_element_type=jnp.float32) * dinv


def _tc_out_body(dinv_ref, acc_ref, g_ref, b_ref, o_ref):
    dinv = dinv_ref[...]
    o_ref[...] = (acc_ref[0] + acc_ref[1] + g_ref[...]) * dinv + b_ref[...]


_dinv_spec = pl.BlockSpec((_R, 1), lambda i: (i, 0))
_row_spec = pl.BlockSpec((_R, D), lambda i: (i, 0))
_deg_spec = pl.BlockSpec((_R, NW), lambda i: (i, 0))
_acc_spec = pl.BlockSpec((NC, _R, D), lambda i: (0, i, 0))
_w_spec = pl.BlockSpec((D, D), lambda i: (0, 0))
_b_spec = pl.BlockSpec((1, D), lambda i: (0, 0))
_out_nd = jax.ShapeDtypeStruct((N, D), jnp.float32)

_tc_hg = pl.pallas_call(
    _tc_hg_body, grid=(_GRID,),
    in_specs=[_deg_spec, _row_spec, _w_spec],
    out_specs=[_row_spec, _dinv_spec],
    out_shape=[_out_nd, jax.ShapeDtypeStruct((N, 1), jnp.float32)])

_tc_mid = pl.pallas_call(
    _tc_mid_body, grid=(_GRID,),
    in_specs=[_dinv_spec, _acc_spec, _row_spec, _b_spec, _w_spec],
    out_specs=_row_spec, out_shape=_out_nd)

_tc_out = pl.pallas_call(
    _tc_out_body, grid=(_GRID,),
    in_specs=[_dinv_spec, _acc_spec, _row_spec, _b_spec],
    out_specs=_row_spec, out_shape=_out_nd)


def kernel(x, edge_index, W1, b1, W2, b2):
    src = edge_index[0].astype(jnp.int32).reshape(NW, NCH, CHUNK)
    dst = edge_index[1].astype(jnp.int32).reshape(NW, NCH, CHUNK)
    idx = jnp.stack([src, dst], axis=2)
    dst_flat = edge_index[1].astype(jnp.int32).reshape(NW, EPW)

    deg_parts = _sc_degree(dst_flat)
    b1r = b1.reshape(1, D)
    b2r = b2.reshape(1, D)

    g1, dinv = _tc_hg(deg_parts.T, x, W1)
    acc1 = _sc_edge_accum(idx, g1)
    g2 = _tc_mid(dinv, acc1, g1, b1r, W2)
    acc2 = _sc_edge_accum(idx, g2)
    out = _tc_out(dinv, acc2, g2, b2r)
    return out
